# speedup vs baseline: 1.5096x; 1.5096x over previous
import functools

import jax
import jax.numpy as jnp
from jax.experimental import pallas as pl
from jax.experimental.pallas import tpu as pltpu

_B, _C, _N, _CL, _K = 4, 128, 10000, 256, 6
_T = 2048
_NPAD = ((_N + _T - 1) // _T) * _T
_NT = _NPAD // _T
_NEG = float("-inf")


def _stats_body(x_ref, out_ref):
    nt = pl.program_id(1)

    @pl.when(nt == 0)
    def _():
        out_ref[...] = jnp.zeros_like(out_ref)

    x = x_ref[0]
    out_ref[0, :, 0:1] += jnp.sum(x, axis=1, keepdims=True)
    out_ref[0, :, 1:2] += jnp.sum(x * x, axis=1, keepdims=True)


def _pool_body(x_ref, coef_ref, wd_ref, out_ref, m_ref, s_ref, u_ref):
    nt = pl.program_id(1)

    @pl.when(nt == 0)
    def _():
        m_ref[...] = jnp.full_like(m_ref, _NEG)
        s_ref[...] = jnp.zeros_like(s_ref)
        u_ref[...] = jnp.zeros_like(u_ref)

    x = x_ref[0]
    alpha = coef_ref[0, :, 0:1]
    delta = coef_ref[0, :, 1:2]
    h = jnp.maximum(alpha * x + delta, 0.0)
    e = jnp.dot(wd_ref[...], h, preferred_element_type=jnp.float32)
    col = jax.lax.broadcasted_iota(jnp.int32, e.shape, 1) + nt * _T
    e = jnp.where(col < _N, e, _NEG)
    m_old = m_ref[...]
    m_new = jnp.maximum(m_old, jnp.max(e, axis=1, keepdims=True))
    scale = jnp.exp(m_old - m_new)
    p = jnp.exp(e - m_new)
    s_ref[...] = s_ref[...] * scale + jnp.sum(p, axis=1, keepdims=True)
    u_ref[...] = u_ref[...] * scale + jax.lax.dot_general(
        p, x, (((1,), (1,)), ((), ())), preferred_element_type=jnp.float32)
    m_ref[...] = m_new

    @pl.when(nt == _NT - 1)
    def _():
        out_ref[0] = u_ref[...] / s_ref[...]


def _dgcnn_body(xds_ref, m1_ref, a2_ref, wg2t_ref, b1_ref, b2_ref, ws2_ref,
                out_ref):
    x = xds_ref[0]
    g_inner = jax.lax.dot_general(
        x, x, (((0,), (0,)), ((), ())), preferred_element_type=jnp.float32)
    rows = jax.lax.broadcasted_iota(jnp.int32, (_C, _C), 0)
    cols = jax.lax.broadcasted_iota(jnp.int32, (_C, _C), 1)
    eye = (rows == cols).astype(jnp.float32)
    d_col = jnp.sum(g_inner * eye, axis=1, keepdims=True)
    d_row = jnp.sum(x * x, axis=0, keepdims=True)
    pd = 2.0 * g_inner - d_col - d_row

    p_mat = jax.lax.dot_general(
        x, m1_ref[...], (((0,), (0,)), ((), ())),
        preferred_element_type=jnp.float32)
    q_mat = jax.lax.dot_general(
        x, a2_ref[...], (((0,), (0,)), ((), ())),
        preferred_element_type=jnp.float32)

    b1 = b1_ref[...]
    b2 = b2_ref[...]
    work = pd
    gmax = jnp.full((_C, _CL), _NEG, jnp.float32)
    for _ in range(_K):
        m = jnp.max(work, axis=1, keepdims=True)
        cand = jnp.where(work == m, cols, jnp.int32(1 << 30))
        sel = jnp.min(cand, axis=1, keepdims=True)
        hit = cols == sel
        onehot = hit.astype(jnp.float32)
        f_q = jnp.dot(onehot, q_mat, preferred_element_type=jnp.float32)
        g1 = jnp.maximum(p_mat - f_q + b1, 0.0)
        g2 = jnp.maximum(
            jnp.dot(g1, wg2t_ref[...], preferred_element_type=jnp.float32)
            + b2, 0.0)
        gmax = jnp.maximum(gmax, g2)
        work = jnp.where(hit, _NEG, work)
    out_ref[0] = jnp.dot(ws2_ref[...], gmax, preferred_element_type=jnp.float32)


def _unpool_body(x_ref, coef_ref, wu_ref, bu_ref, a_ref, ws1_ref, bs_ref,
                 out_ref):
    x = x_ref[0]
    alpha = coef_ref[0, :, 0:1]
    delta = coef_ref[0, :, 1:2]
    h = jnp.maximum(alpha * x + delta, 0.0)
    e = jnp.dot(wu_ref[...], h, preferred_element_type=jnp.float32)
    e = e + bu_ref[...]
    m = jnp.max(e, axis=0, keepdims=True)
    p = jnp.exp(e - m)
    s2 = p / jnp.sum(p, axis=0, keepdims=True)
    out_ref[0] = (jnp.dot(ws1_ref[...], x, preferred_element_type=jnp.float32)
                  + jnp.dot(a_ref[0], s2, preferred_element_type=jnp.float32)
                  + bs_ref[...])


def kernel(data, bn_d_gamma, bn_d_beta, W_d, b_d, bn_u_gamma, bn_u_beta, W_u,
           b_u, W_g1, b_g1, bn_g1_gamma, bn_g1_beta, W_g2, b_g2, bn_g2_gamma,
           bn_g2_beta, W_s, b_s):
    f32 = jnp.float32
    xpad = jnp.pad(data[..., 0], ((0, 0), (0, 0), (0, _NPAD - _N)))

    grid_n = (_B, _NT)
    tile_spec = pl.BlockSpec((1, _C, _T), lambda b, n: (b, 0, n))
    coef_spec = pl.BlockSpec((1, _C, 2), lambda b, n: (b, 0, 0))
    seq2 = pltpu.CompilerParams(
        dimension_semantics=("arbitrary", "arbitrary"))

    stats = pl.pallas_call(
        _stats_body,
        grid=grid_n,
        in_specs=[tile_spec],
        out_specs=pl.BlockSpec((1, _C, 2), lambda b, n: (b, 0, 0)),
        out_shape=jax.ShapeDtypeStruct((_B, _C, 2), f32),
        compiler_params=seq2,
    )(xpad)

    mean = stats[..., 0] / _N
    var = stats[..., 1] / _N - mean * mean
    bn_scale = 1.0 / jnp.sqrt(jnp.float32(1.0 + 1e-5))

    def coefs(gamma, beta):
        a = gamma[None, :] * bn_scale / jnp.sqrt(var + 1e-3)
        d = beta[None, :] - a * mean
        return jnp.stack([a, d], axis=-1).astype(f32)

    coef_d = coefs(bn_d_gamma, bn_d_beta)
    coef_u = coefs(bn_u_gamma, bn_u_beta)

    xds = pl.pallas_call(
        _pool_body,
        grid=grid_n,
        in_specs=[
            tile_spec,
            coef_spec,
            pl.BlockSpec((_CL, _C), lambda b, n: (0, 0)),
        ],
        out_specs=pl.BlockSpec((1, _CL, _C), lambda b, n: (b, 0, 0)),
        out_shape=jax.ShapeDtypeStruct((_B, _CL, _C), f32),
        scratch_shapes=[
            pltpu.VMEM((_CL, 1), f32),
            pltpu.VMEM((_CL, 1), f32),
            pltpu.VMEM((_CL, _C), f32),
        ],
        compiler_params=seq2,
    )(xpad, coef_d, W_d)

    s1 = (bn_g1_gamma * bn_scale)
    s2 = (bn_g2_gamma * bn_scale)
    wg1t = (W_g1 * s1[:, None]).T
    b1row = (b_g1 * s1 + bn_g1_beta)[None, :]
    wg2t = (W_g2 * s2[:, None]).T
    b2row = (b_g2 * s2 + bn_g2_beta)[None, :]
    m1 = wg1t[:_CL] + wg1t[_CL:]
    a2 = wg1t[_CL:]

    a_mat = pl.pallas_call(
        _dgcnn_body,
        grid=(_B,),
        in_specs=[
            pl.BlockSpec((1, _CL, _C), lambda b: (b, 0, 0)),
            pl.BlockSpec((_CL, _CL), lambda b: (0, 0)),
            pl.BlockSpec((_CL, _CL), lambda b: (0, 0)),
            pl.BlockSpec((_CL, _CL), lambda b: (0, 0)),
            pl.BlockSpec((1, _CL), lambda b: (0, 0)),
            pl.BlockSpec((1, _CL), lambda b: (0, 0)),
            pl.BlockSpec((_C, _C), lambda b: (0, 0)),
        ],
        out_specs=pl.BlockSpec((1, _C, _CL), lambda b: (b, 0, 0)),
        out_shape=jax.ShapeDtypeStruct((_B, _C, _CL), f32),
        compiler_params=pltpu.CompilerParams(
            dimension_semantics=("arbitrary",)),
    )(xds, m1, a2, wg2t, b1row, b2row, W_s[:, _C:])

    outp = pl.pallas_call(
        _unpool_body,
        grid=grid_n,
        in_specs=[
            tile_spec,
            coef_spec,
            pl.BlockSpec((_CL, _C), lambda b, n: (0, 0)),
            pl.BlockSpec((_CL, 1), lambda b, n: (0, 0)),
            pl.BlockSpec((1, _C, _CL), lambda b, n: (b, 0, 0)),
            pl.BlockSpec((_C, _C), lambda b, n: (0, 0)),
            pl.BlockSpec((_C, 1), lambda b, n: (0, 0)),
        ],
        out_specs=tile_spec,
        out_shape=jax.ShapeDtypeStruct((_B, _C, _NPAD), f32),
        compiler_params=seq2,
    )(xpad, coef_u, W_u, b_u[:, None], a_mat, W_s[:, :_C], b_s[:, None])

    return outp[:, :, :_N, None]


# speedup vs baseline: 1.8186x; 1.2046x over previous
import functools

import jax
import jax.numpy as jnp
from jax.experimental import pallas as pl
from jax.experimental.pallas import tpu as pltpu

_B, _C, _N, _CL, _K = 4, 128, 10000, 256, 6
_T = 2048
_NPAD = ((_N + _T - 1) // _T) * _T
_NT = _NPAD // _T
_NEG = float("-inf")


def _lane_mask(nt):
    col = jax.lax.broadcasted_iota(jnp.int32, (_C, _T), 1) + nt * _T
    return col < _N


def _stats_body(x_ref, out_ref):
    nt = pl.program_id(1)

    @pl.when(nt == 0)
    def _():
        out_ref[...] = jnp.zeros_like(out_ref)

    x = jnp.where(_lane_mask(nt), x_ref[0], 0.0)
    out_ref[0, :, 0:1] += jnp.sum(x, axis=1, keepdims=True)
    out_ref[0, :, 1:2] += jnp.sum(x * x, axis=1, keepdims=True)


def _pool_body(x_ref, coef_ref, wd_ref, out_ref, m_ref, s_ref, u_ref):
    nt = pl.program_id(1)

    @pl.when(nt == 0)
    def _():
        m_ref[...] = jnp.full_like(m_ref, _NEG)
        s_ref[...] = jnp.zeros_like(s_ref)
        u_ref[...] = jnp.zeros_like(u_ref)

    valid = _lane_mask(nt)
    x = jnp.where(valid, x_ref[0], 0.0)
    alpha = coef_ref[0, :, 0:1]
    delta = coef_ref[0, :, 1:2]
    h = jnp.maximum(alpha * x + delta, 0.0)
    e = jnp.dot(wd_ref[...], h, preferred_element_type=jnp.float32)
    col = jax.lax.broadcasted_iota(jnp.int32, e.shape, 1) + nt * _T
    e = jnp.where(col < _N, e, _NEG)
    m_old = m_ref[...]
    m_new = jnp.maximum(m_old, jnp.max(e, axis=1, keepdims=True))
    scale = jnp.exp(m_old - m_new)
    p = jnp.exp(e - m_new)
    s_ref[...] = s_ref[...] * scale + jnp.sum(p, axis=1, keepdims=True)
    u_ref[...] = u_ref[...] * scale + jax.lax.dot_general(
        p, x, (((1,), (1,)), ((), ())), preferred_element_type=jnp.float32)
    m_ref[...] = m_new

    @pl.when(nt == _NT - 1)
    def _():
        out_ref[0] = u_ref[...] / s_ref[...]


def _dgcnn_body(xds_ref, m1_ref, a2_ref, wg2t_ref, b1_ref, b2_ref, ws2_ref,
                out_ref):
    x = xds_ref[0]
    g_inner = jax.lax.dot_general(
        x, x, (((0,), (0,)), ((), ())), preferred_element_type=jnp.float32)
    rows = jax.lax.broadcasted_iota(jnp.int32, (_C, _C), 0)
    cols = jax.lax.broadcasted_iota(jnp.int32, (_C, _C), 1)
    eye = (rows == cols).astype(jnp.float32)
    d_col = jnp.sum(g_inner * eye, axis=1, keepdims=True)
    d_row = jnp.sum(x * x, axis=0, keepdims=True)
    pd = 2.0 * g_inner - d_col - d_row

    p_mat = jax.lax.dot_general(
        x, m1_ref[...], (((0,), (0,)), ((), ())),
        preferred_element_type=jnp.float32)
    q_mat = jax.lax.dot_general(
        x, a2_ref[...], (((0,), (0,)), ((), ())),
        preferred_element_type=jnp.float32)

    b1 = b1_ref[...]
    b2 = b2_ref[...]
    work = pd
    gmax = jnp.full((_C, _CL), _NEG, jnp.float32)
    for _ in range(_K):
        m = jnp.max(work, axis=1, keepdims=True)
        cand = jnp.where(work == m, cols, jnp.int32(1 << 30))
        sel = jnp.min(cand, axis=1, keepdims=True)
        hit = cols == sel
        onehot = hit.astype(jnp.float32)
        f_q = jnp.dot(onehot, q_mat, preferred_element_type=jnp.float32)
        g1 = jnp.maximum(p_mat - f_q + b1, 0.0)
        g2 = jnp.maximum(
            jnp.dot(g1, wg2t_ref[...], preferred_element_type=jnp.float32)
            + b2, 0.0)
        gmax = jnp.maximum(gmax, g2)
        work = jnp.where(hit, _NEG, work)
    out_ref[0] = jnp.dot(ws2_ref[...], gmax, preferred_element_type=jnp.float32)


def _unpool_body(x_ref, coef_ref, wu_ref, bu_ref, a_ref, ws1_ref, bs_ref,
                 out_ref):
    nt = pl.program_id(1)
    x = jnp.where(_lane_mask(nt), x_ref[0], 0.0)
    alpha = coef_ref[0, :, 0:1]
    delta = coef_ref[0, :, 1:2]
    h = jnp.maximum(alpha * x + delta, 0.0)
    e = jnp.dot(wu_ref[...], h, preferred_element_type=jnp.float32)
    e = e + bu_ref[...]
    m = jnp.max(e, axis=0, keepdims=True)
    p = jnp.exp(e - m)
    s2 = p / jnp.sum(p, axis=0, keepdims=True)
    out_ref[0] = (jnp.dot(ws1_ref[...], x, preferred_element_type=jnp.float32)
                  + jnp.dot(a_ref[0], s2, preferred_element_type=jnp.float32)
                  + bs_ref[...])


def kernel(data, bn_d_gamma, bn_d_beta, W_d, b_d, bn_u_gamma, bn_u_beta, W_u,
           b_u, W_g1, b_g1, bn_g1_gamma, bn_g1_beta, W_g2, b_g2, bn_g2_gamma,
           bn_g2_beta, W_s, b_s):
    f32 = jnp.float32
    xpad = data[..., 0]

    grid_n = (_B, _NT)
    tile_spec = pl.BlockSpec((1, _C, _T), lambda b, n: (b, 0, n))
    coef_spec = pl.BlockSpec((1, _C, 2), lambda b, n: (b, 0, 0))
    seq2 = pltpu.CompilerParams(
        dimension_semantics=("arbitrary", "arbitrary"))

    stats = pl.pallas_call(
        _stats_body,
        grid=grid_n,
        in_specs=[tile_spec],
        out_specs=pl.BlockSpec((1, _C, 2), lambda b, n: (b, 0, 0)),
        out_shape=jax.ShapeDtypeStruct((_B, _C, 2), f32),
        compiler_params=seq2,
    )(xpad)

    mean = stats[..., 0] / _N
    var = stats[..., 1] / _N - mean * mean
    bn_scale = 1.0 / jnp.sqrt(jnp.float32(1.0 + 1e-5))

    def coefs(gamma, beta):
        a = gamma[None, :] * bn_scale / jnp.sqrt(var + 1e-3)
        d = beta[None, :] - a * mean
        return jnp.stack([a, d], axis=-1).astype(f32)

    coef_d = coefs(bn_d_gamma, bn_d_beta)
    coef_u = coefs(bn_u_gamma, bn_u_beta)

    xds = pl.pallas_call(
        _pool_body,
        grid=grid_n,
        in_specs=[
            tile_spec,
            coef_spec,
            pl.BlockSpec((_CL, _C), lambda b, n: (0, 0)),
        ],
        out_specs=pl.BlockSpec((1, _CL, _C), lambda b, n: (b, 0, 0)),
        out_shape=jax.ShapeDtypeStruct((_B, _CL, _C), f32),
        scratch_shapes=[
            pltpu.VMEM((_CL, 1), f32),
            pltpu.VMEM((_CL, 1), f32),
            pltpu.VMEM((_CL, _C), f32),
        ],
        compiler_params=seq2,
    )(xpad, coef_d, W_d)

    s1 = (bn_g1_gamma * bn_scale)
    s2 = (bn_g2_gamma * bn_scale)
    wg1t = (W_g1 * s1[:, None]).T
    b1row = (b_g1 * s1 + bn_g1_beta)[None, :]
    wg2t = (W_g2 * s2[:, None]).T
    b2row = (b_g2 * s2 + bn_g2_beta)[None, :]
    m1 = wg1t[:_CL] + wg1t[_CL:]
    a2 = wg1t[_CL:]

    a_mat = pl.pallas_call(
        _dgcnn_body,
        grid=(_B,),
        in_specs=[
            pl.BlockSpec((1, _CL, _C), lambda b: (b, 0, 0)),
            pl.BlockSpec((_CL, _CL), lambda b: (0, 0)),
            pl.BlockSpec((_CL, _CL), lambda b: (0, 0)),
            pl.BlockSpec((_CL, _CL), lambda b: (0, 0)),
            pl.BlockSpec((1, _CL), lambda b: (0, 0)),
            pl.BlockSpec((1, _CL), lambda b: (0, 0)),
            pl.BlockSpec((_C, _C), lambda b: (0, 0)),
        ],
        out_specs=pl.BlockSpec((1, _C, _CL), lambda b: (b, 0, 0)),
        out_shape=jax.ShapeDtypeStruct((_B, _C, _CL), f32),
        compiler_params=pltpu.CompilerParams(
            dimension_semantics=("arbitrary",)),
    )(xds, m1, a2, wg2t, b1row, b2row, W_s[:, _C:])

    outp = pl.pallas_call(
        _unpool_body,
        grid=grid_n,
        in_specs=[
            tile_spec,
            coef_spec,
            pl.BlockSpec((_CL, _C), lambda b, n: (0, 0)),
            pl.BlockSpec((_CL, 1), lambda b, n: (0, 0)),
            pl.BlockSpec((1, _C, _CL), lambda b, n: (b, 0, 0)),
            pl.BlockSpec((_C, _C), lambda b, n: (0, 0)),
            pl.BlockSpec((_C, 1), lambda b, n: (0, 0)),
        ],
        out_specs=tile_spec,
        out_shape=jax.ShapeDtypeStruct((_B, _C, _N), f32),
        compiler_params=seq2,
    )(xpad, coef_u, W_u, b_u[:, None], a_mat, W_s[:, :_C], b_s[:, None])

    return outp[..., None]


# speedup vs baseline: 1.8373x; 1.0103x over previous
import jax
import jax.numpy as jnp
from jax.experimental import pallas as pl
from jax.experimental.pallas import tpu as pltpu

_B, _C, _N, _CL, _K = 4, 128, 10000, 256, 6
_T = 2048
_NT = (_N + _T - 1) // _T
_NEG = float("-inf")
_BN_S = 0.9999950000374997


def _lane_mask(nt):
    col = jax.lax.broadcasted_iota(jnp.int32, (_C, _T), 1) + nt * _T
    return col < _N


def _coefs(stats_ref, b, g_ref, be_ref):
    mean = stats_ref[b, :, 0:1] * (1.0 / _N)
    var = stats_ref[b, :, 1:2] * (1.0 / _N) - mean * mean
    a = g_ref[...] * _BN_S * jax.lax.rsqrt(var + 1e-3)
    d = be_ref[...] - a * mean
    return a, d


def _body(x_ref, gd_ref, bd_ref, gu_ref, beu_ref, wd_ref, wu_ref, bu_ref,
          m1_ref, a2_ref, wg2t_ref, b1_ref, b2_ref, ws1_ref, ws2_ref, bs_ref,
          out_ref, stats_ref, m_ref, s_ref, u_ref, xds_ref, a_ref):
    ph = pl.program_id(0)
    b = pl.program_id(1)
    nt = pl.program_id(2)

    @pl.when(ph == 0)
    def _stats():
        x = jnp.where(_lane_mask(nt), x_ref[0], 0.0)
        s0 = jnp.sum(x, axis=1, keepdims=True)
        s1 = jnp.sum(x * x, axis=1, keepdims=True)

        @pl.when(nt == 0)
        def _():
            stats_ref[b, :, 0:1] = s0
            stats_ref[b, :, 1:2] = s1

        @pl.when(nt > 0)
        def _():
            stats_ref[b, :, 0:1] += s0
            stats_ref[b, :, 1:2] += s1

    @pl.when(ph == 1)
    def _pool():
        @pl.when(nt == 0)
        def _():
            m_ref[...] = jnp.full_like(m_ref, _NEG)
            s_ref[...] = jnp.zeros_like(s_ref)
            u_ref[...] = jnp.zeros_like(u_ref)

        valid = _lane_mask(nt)
        x = jnp.where(valid, x_ref[0], 0.0)
        alpha, delta = _coefs(stats_ref, b, gd_ref, bd_ref)
        h = jnp.maximum(alpha * x + delta, 0.0)
        e = jnp.dot(wd_ref[...], h, preferred_element_type=jnp.float32)
        e = jnp.where(jax.lax.broadcasted_iota(jnp.int32, e.shape, 1)
                      + nt * _T < _N, e, _NEG)
        m_old = m_ref[...]
        m_new = jnp.maximum(m_old, jnp.max(e, axis=1, keepdims=True))
        scale = jnp.exp(m_old - m_new)
        p = jnp.exp(e - m_new)
        s_ref[...] = s_ref[...] * scale + jnp.sum(p, axis=1, keepdims=True)
        u_ref[...] = u_ref[...] * scale + jax.lax.dot_general(
            p, x, (((1,), (1,)), ((), ())), preferred_element_type=jnp.float32)
        m_ref[...] = m_new

        @pl.when(nt == _NT - 1)
        def _():
            xds_ref[b] = u_ref[...] / s_ref[...]

    @pl.when(ph == 2)
    def _unpool():
        @pl.when(nt == 0)
        def _dgcnn():
            xv = xds_ref[b]
            g_inner = jax.lax.dot_general(
                xv, xv, (((0,), (0,)), ((), ())),
                preferred_element_type=jnp.float32)
            rows = jax.lax.broadcasted_iota(jnp.int32, (_C, _C), 0)
            cols = jax.lax.broadcasted_iota(jnp.int32, (_C, _C), 1)
            eye = (rows == cols).astype(jnp.float32)
            d_col = jnp.sum(g_inner * eye, axis=1, keepdims=True)
            d_row = jnp.sum(xv * xv, axis=0, keepdims=True)
            pd = 2.0 * g_inner - d_col - d_row
            p_mat = jax.lax.dot_general(
                xv, m1_ref[...], (((0,), (0,)), ((), ())),
                preferred_element_type=jnp.float32)
            q_mat = jax.lax.dot_general(
                xv, a2_ref[...], (((0,), (0,)), ((), ())),
                preferred_element_type=jnp.float32)
            b1 = b1_ref[...]
            b2 = b2_ref[...]
            work = pd
            gmax = jnp.full((_C, _CL), _NEG, jnp.float32)
            for _ in range(_K):
                m = jnp.max(work, axis=1, keepdims=True)
                cand = jnp.where(work == m, cols, jnp.int32(1 << 30))
                sel = jnp.min(cand, axis=1, keepdims=True)
                hit = cols == sel
                onehot = hit.astype(jnp.float32)
                f_q = jnp.dot(onehot, q_mat, preferred_element_type=jnp.float32)
                g1 = jnp.maximum(p_mat - f_q + b1, 0.0)
                g2 = jnp.maximum(
                    jnp.dot(g1, wg2t_ref[...],
                            preferred_element_type=jnp.float32) + b2, 0.0)
                gmax = jnp.maximum(gmax, g2)
                work = jnp.where(hit, _NEG, work)
            a_ref[...] = jnp.dot(ws2_ref[...], gmax,
                                 preferred_element_type=jnp.float32)

        x = jnp.where(_lane_mask(nt), x_ref[0], 0.0)
        alpha, delta = _coefs(stats_ref, b, gu_ref, beu_ref)
        h = jnp.maximum(alpha * x + delta, 0.0)
        e = jnp.dot(wu_ref[...], h, preferred_element_type=jnp.float32)
        e = e + bu_ref[...]
        m = jnp.max(e, axis=0, keepdims=True)
        p = jnp.exp(e - m)
        s2 = p / jnp.sum(p, axis=0, keepdims=True)
        out_ref[0] = (
            jnp.dot(ws1_ref[...], x, preferred_element_type=jnp.float32)
            + jnp.dot(a_ref[...], s2, preferred_element_type=jnp.float32)
            + bs_ref[...])


def kernel(data, bn_d_gamma, bn_d_beta, W_d, b_d, bn_u_gamma, bn_u_beta, W_u,
           b_u, W_g1, b_g1, bn_g1_gamma, bn_g1_beta, W_g2, b_g2, bn_g2_gamma,
           bn_g2_beta, W_s, b_s):
    f32 = jnp.float32
    x3 = data[..., 0]

    s1 = bn_g1_gamma * _BN_S
    s2 = bn_g2_gamma * _BN_S
    wg1t = (W_g1 * s1[:, None]).T
    b1row = (b_g1 * s1 + bn_g1_beta)[None, :]
    wg2t = (W_g2 * s2[:, None]).T
    b2row = (b_g2 * s2 + bn_g2_beta)[None, :]
    m1 = wg1t[:_CL] + wg1t[_CL:]
    a2 = wg1t[_CL:]

    def full(shape):
        nd = len(shape)
        return pl.BlockSpec(shape, lambda p, b, n, _nd=nd: (0,) * _nd)

    tile_spec = pl.BlockSpec((1, _C, _T), lambda p, b, n: (b, 0, n))
    out_spec = pl.BlockSpec(
        (1, _C, _T),
        lambda p, b, n: (jnp.where(p == 2, b, 0), 0, jnp.where(p == 2, n, 0)))

    outp = pl.pallas_call(
        _body,
        grid=(3, _B, _NT),
        in_specs=[
            tile_spec,
            full((_C, 1)), full((_C, 1)), full((_C, 1)), full((_C, 1)),
            full((_CL, _C)), full((_CL, _C)), full((_CL, 1)),
            full((_CL, _CL)), full((_CL, _CL)), full((_CL, _CL)),
            full((1, _CL)), full((1, _CL)),
            full((_C, _C)), full((_C, _C)), full((_C, 1)),
        ],
        out_specs=out_spec,
        out_shape=jax.ShapeDtypeStruct((_B, _C, _N), f32),
        scratch_shapes=[
            pltpu.VMEM((_B, _C, 2), f32),
            pltpu.VMEM((_CL, 1), f32),
            pltpu.VMEM((_CL, 1), f32),
            pltpu.VMEM((_CL, _C), f32),
            pltpu.VMEM((_B, _CL, _C), f32),
            pltpu.VMEM((_C, _CL), f32),
        ],
        compiler_params=pltpu.CompilerParams(
            dimension_semantics=("arbitrary", "arbitrary", "arbitrary")),
    )(x3,
      bn_d_gamma[:, None], bn_d_beta[:, None],
      bn_u_gamma[:, None], bn_u_beta[:, None],
      W_d, W_u, b_u[:, None],
      m1, a2, wg2t, b1row, b2row,
      W_s[:, :_C], W_s[:, _C:], b_s[:, None])

    return outp[..., None]


# speedup vs baseline: 1.9407x; 1.0563x over previous
import jax
import jax.numpy as jnp
from jax.experimental import pallas as pl
from jax.experimental.pallas import tpu as pltpu

_B, _C, _N, _CL, _K = 4, 128, 10000, 256, 6
_T = 2560
_NT = (_N + _T - 1) // _T
_NEG = float("-inf")
_BN_S = 0.9999950000374997


def _lane_mask(nt):
    col = jax.lax.broadcasted_iota(jnp.int32, (_C, _T), 1) + nt * _T
    return col < _N


def _coefs(stats_ref, b, g_ref, be_ref):
    mean = stats_ref[b, :, 0:1] * (1.0 / _N)
    var = stats_ref[b, :, 1:2] * (1.0 / _N) - mean * mean
    a = g_ref[...] * _BN_S * jax.lax.rsqrt(var + 1e-3)
    d = be_ref[...] - a * mean
    return a, d


def _body(x_ref, gd_ref, bd_ref, gu_ref, beu_ref, wd_ref, wu_ref, bu_ref,
          m1_ref, a2_ref, wg2t_ref, b1_ref, b2_ref, ws1_ref, ws2_ref, bs_ref,
          out_ref, stats_ref, m_ref, s_ref, u_ref, xds_ref, a_ref):
    ph = pl.program_id(0)
    b = pl.program_id(1)
    nt = pl.program_id(2)

    @pl.when(ph == 0)
    def _stats():
        x = jnp.where(_lane_mask(nt), x_ref[0], 0.0)
        s0 = jnp.sum(x, axis=1, keepdims=True)
        s1 = jnp.sum(x * x, axis=1, keepdims=True)

        @pl.when(nt == 0)
        def _():
            stats_ref[b, :, 0:1] = s0
            stats_ref[b, :, 1:2] = s1

        @pl.when(nt > 0)
        def _():
            stats_ref[b, :, 0:1] += s0
            stats_ref[b, :, 1:2] += s1

    @pl.when(ph == 1)
    def _pool():
        @pl.when(nt == 0)
        def _():
            m_ref[...] = jnp.full_like(m_ref, _NEG)
            s_ref[...] = jnp.zeros_like(s_ref)
            u_ref[...] = jnp.zeros_like(u_ref)

        valid = _lane_mask(nt)
        x = jnp.where(valid, x_ref[0], 0.0)
        alpha, delta = _coefs(stats_ref, b, gd_ref, bd_ref)
        h = jnp.maximum(alpha * x + delta, 0.0)
        e = jnp.dot(wd_ref[...], h, preferred_element_type=jnp.float32)
        e = jnp.where(jax.lax.broadcasted_iota(jnp.int32, e.shape, 1)
                      + nt * _T < _N, e, _NEG)
        m_old = m_ref[...]
        m_new = jnp.maximum(m_old, jnp.max(e, axis=1, keepdims=True))
        scale = jnp.exp(m_old - m_new)
        p = jnp.exp(e - m_new)
        s_ref[...] = s_ref[...] * scale + jnp.sum(p, axis=1, keepdims=True)
        u_ref[...] = u_ref[...] * scale + jax.lax.dot_general(
            p, x, (((1,), (1,)), ((), ())), preferred_element_type=jnp.float32)
        m_ref[...] = m_new

        @pl.when(nt == _NT - 1)
        def _():
            xds_ref[b] = u_ref[...] / s_ref[...]

    @pl.when(ph == 2)
    def _unpool():
        @pl.when(nt == 0)
        def _dgcnn():
            xv = xds_ref[b]
            g_inner = jax.lax.dot_general(
                xv, xv, (((0,), (0,)), ((), ())),
                preferred_element_type=jnp.float32)
            rows = jax.lax.broadcasted_iota(jnp.int32, (_C, _C), 0)
            cols = jax.lax.broadcasted_iota(jnp.int32, (_C, _C), 1)
            eye = (rows == cols).astype(jnp.float32)
            d_col = jnp.sum(g_inner * eye, axis=1, keepdims=True)
            d_row = jnp.sum(xv * xv, axis=0, keepdims=True)
            pd = 2.0 * g_inner - d_col - d_row
            p_mat = jax.lax.dot_general(
                xv, m1_ref[...], (((0,), (0,)), ((), ())),
                preferred_element_type=jnp.float32)
            q_mat = jax.lax.dot_general(
                xv, a2_ref[...], (((0,), (0,)), ((), ())),
                preferred_element_type=jnp.float32)
            b1 = b1_ref[...]
            b2 = b2_ref[...]
            work = pd
            gmax = jnp.full((_C, _CL), _NEG, jnp.float32)
            for _ in range(_K):
                m = jnp.max(work, axis=1, keepdims=True)
                cand = jnp.where(work == m, cols, jnp.int32(1 << 30))
                sel = jnp.min(cand, axis=1, keepdims=True)
                hit = cols == sel
                onehot = hit.astype(jnp.float32)
                f_q = jnp.dot(onehot, q_mat, preferred_element_type=jnp.float32)
                g1 = jnp.maximum(p_mat - f_q + b1, 0.0)
                g2 = jnp.maximum(
                    jnp.dot(g1, wg2t_ref[...],
                            preferred_element_type=jnp.float32) + b2, 0.0)
                gmax = jnp.maximum(gmax, g2)
                work = jnp.where(hit, _NEG, work)
            a_ref[...] = jnp.dot(ws2_ref[...], gmax,
                                 preferred_element_type=jnp.float32)

        x = jnp.where(_lane_mask(nt), x_ref[0], 0.0)
        alpha, delta = _coefs(stats_ref, b, gu_ref, beu_ref)
        h = jnp.maximum(alpha * x + delta, 0.0)
        e = jnp.dot(wu_ref[...], h, preferred_element_type=jnp.float32)
        e = e + bu_ref[...]
        m = jnp.max(e, axis=0, keepdims=True)
        p = jnp.exp(e - m)
        s2 = p / jnp.sum(p, axis=0, keepdims=True)
        out_ref[0] = (
            jnp.dot(ws1_ref[...], x, preferred_element_type=jnp.float32)
            + jnp.dot(a_ref[...], s2, preferred_element_type=jnp.float32)
            + bs_ref[...])


def kernel(data, bn_d_gamma, bn_d_beta, W_d, b_d, bn_u_gamma, bn_u_beta, W_u,
           b_u, W_g1, b_g1, bn_g1_gamma, bn_g1_beta, W_g2, b_g2, bn_g2_gamma,
           bn_g2_beta, W_s, b_s):
    f32 = jnp.float32
    x3 = data[..., 0]

    s1 = bn_g1_gamma * _BN_S
    s2 = bn_g2_gamma * _BN_S
    wg1t = (W_g1 * s1[:, None]).T
    b1row = (b_g1 * s1 + bn_g1_beta)[None, :]
    wg2t = (W_g2 * s2[:, None]).T
    b2row = (b_g2 * s2 + bn_g2_beta)[None, :]
    m1 = wg1t[:_CL] + wg1t[_CL:]
    a2 = wg1t[_CL:]

    def full(shape):
        nd = len(shape)
        return pl.BlockSpec(shape, lambda p, b, n, _nd=nd: (0,) * _nd)

    tile_spec = pl.BlockSpec((1, _C, _T), lambda p, b, n: (b, 0, n))
    out_spec = pl.BlockSpec(
        (1, _C, _T),
        lambda p, b, n: (jnp.where(p == 2, b, 0), 0, jnp.where(p == 2, n, 0)))

    outp = pl.pallas_call(
        _body,
        grid=(3, _B, _NT),
        in_specs=[
            tile_spec,
            full((_C, 1)), full((_C, 1)), full((_C, 1)), full((_C, 1)),
            full((_CL, _C)), full((_CL, _C)), full((_CL, 1)),
            full((_CL, _CL)), full((_CL, _CL)), full((_CL, _CL)),
            full((1, _CL)), full((1, _CL)),
            full((_C, _C)), full((_C, _C)), full((_C, 1)),
        ],
        out_specs=out_spec,
        out_shape=jax.ShapeDtypeStruct((_B, _C, _N), f32),
        scratch_shapes=[
            pltpu.VMEM((_B, _C, 2), f32),
            pltpu.VMEM((_CL, 1), f32),
            pltpu.VMEM((_CL, 1), f32),
            pltpu.VMEM((_CL, _C), f32),
            pltpu.VMEM((_B, _CL, _C), f32),
            pltpu.VMEM((_C, _CL), f32),
        ],
        compiler_params=pltpu.CompilerParams(
            dimension_semantics=("arbitrary", "arbitrary", "arbitrary")),
    )(x3,
      bn_d_gamma[:, None], bn_d_beta[:, None],
      bn_u_gamma[:, None], bn_u_beta[:, None],
      W_d, W_u, b_u[:, None],
      m1, a2, wg2t, b1row, b2row,
      W_s[:, :_C], W_s[:, _C:], b_s[:, None])

    return outp[..., None]


# speedup vs baseline: 2.1466x; 1.1061x over previous
import jax
import jax.numpy as jnp
from jax.experimental import pallas as pl
from jax.experimental.pallas import tpu as pltpu

_B, _C, _N, _CL, _K = 4, 128, 10000, 256, 6
_T = 5120
_NT = (_N + _T - 1) // _T
_NEG = float("-inf")
_BN_S = 0.9999950000374997


def _lane_mask(nt):
    col = jax.lax.broadcasted_iota(jnp.int32, (_C, _T), 1) + nt * _T
    return col < _N


def _coefs(stats_ref, b, g_ref, be_ref):
    mean = stats_ref[b, :, 0:1] * (1.0 / _N)
    var = stats_ref[b, :, 1:2] * (1.0 / _N) - mean * mean
    a = g_ref[...] * _BN_S * jax.lax.rsqrt(var + 1e-3)
    d = be_ref[...] - a * mean
    return a, d


def _body(x_ref, gd_ref, bd_ref, gu_ref, beu_ref, wd_ref, wu_ref, bu_ref,
          m1_ref, a2_ref, wg2t_ref, b1_ref, b2_ref, ws1_ref, ws2_ref, bs_ref,
          out_ref, stats_ref, m_ref, s_ref, u_ref, xds_ref, a_ref):
    ph = pl.program_id(0)
    b = pl.program_id(1)
    nt = pl.program_id(2)

    @pl.when(ph == 0)
    def _stats():
        x = jnp.where(_lane_mask(nt), x_ref[0], 0.0)
        s0 = jnp.sum(x, axis=1, keepdims=True)
        s1 = jnp.sum(x * x, axis=1, keepdims=True)

        @pl.when(nt == 0)
        def _():
            stats_ref[b, :, 0:1] = s0
            stats_ref[b, :, 1:2] = s1

        @pl.when(nt > 0)
        def _():
            stats_ref[b, :, 0:1] += s0
            stats_ref[b, :, 1:2] += s1

    @pl.when(ph == 1)
    def _pool():
        @pl.when(nt == 0)
        def _():
            m_ref[...] = jnp.full_like(m_ref, _NEG)
            s_ref[...] = jnp.zeros_like(s_ref)
            u_ref[...] = jnp.zeros_like(u_ref)

        valid = _lane_mask(nt)
        x = jnp.where(valid, x_ref[0], 0.0)
        alpha, delta = _coefs(stats_ref, b, gd_ref, bd_ref)
        h = jnp.maximum(alpha * x + delta, 0.0)
        e = jnp.dot(wd_ref[...], h, preferred_element_type=jnp.float32)
        e = jnp.where(jax.lax.broadcasted_iota(jnp.int32, e.shape, 1)
                      + nt * _T < _N, e, _NEG)
        m_old = m_ref[...]
        m_new = jnp.maximum(m_old, jnp.max(e, axis=1, keepdims=True))
        scale = jnp.exp(m_old - m_new)
        p = jnp.exp(e - m_new)
        s_ref[...] = s_ref[...] * scale + jnp.sum(p, axis=1, keepdims=True)
        u_ref[...] = u_ref[...] * scale + jax.lax.dot_general(
            p, x, (((1,), (1,)), ((), ())), preferred_element_type=jnp.float32)
        m_ref[...] = m_new

        @pl.when(nt == _NT - 1)
        def _():
            xds_ref[b] = u_ref[...] / s_ref[...]

    @pl.when(ph == 2)
    def _unpool():
        @pl.when(nt == 0)
        def _dgcnn():
            xv = xds_ref[b]
            g_inner = jax.lax.dot_general(
                xv, xv, (((0,), (0,)), ((), ())),
                preferred_element_type=jnp.float32)
            rows = jax.lax.broadcasted_iota(jnp.int32, (_C, _C), 0)
            cols = jax.lax.broadcasted_iota(jnp.int32, (_C, _C), 1)
            eye = (rows == cols).astype(jnp.float32)
            d_col = jnp.sum(g_inner * eye, axis=1, keepdims=True)
            d_row = jnp.sum(xv * xv, axis=0, keepdims=True)
            pd = 2.0 * g_inner - d_col - d_row
            p_mat = jax.lax.dot_general(
                xv, m1_ref[...], (((0,), (0,)), ((), ())),
                preferred_element_type=jnp.float32)
            q_mat = jax.lax.dot_general(
                xv, a2_ref[...], (((0,), (0,)), ((), ())),
                preferred_element_type=jnp.float32)
            b1 = b1_ref[...]
            b2 = b2_ref[...]
            work = pd
            gmax = jnp.full((_C, _CL), _NEG, jnp.float32)
            for _ in range(_K):
                m = jnp.max(work, axis=1, keepdims=True)
                cand = jnp.where(work == m, cols, jnp.int32(1 << 30))
                sel = jnp.min(cand, axis=1, keepdims=True)
                hit = cols == sel
                onehot = hit.astype(jnp.float32)
                f_q = jnp.dot(onehot, q_mat, preferred_element_type=jnp.float32)
                g1 = jnp.maximum(p_mat - f_q + b1, 0.0)
                g2 = jnp.maximum(
                    jnp.dot(g1, wg2t_ref[...],
                            preferred_element_type=jnp.float32) + b2, 0.0)
                gmax = jnp.maximum(gmax, g2)
                work = jnp.where(hit, _NEG, work)
            a_ref[...] = jnp.dot(ws2_ref[...], gmax,
                                 preferred_element_type=jnp.float32)

        x = jnp.where(_lane_mask(nt), x_ref[0], 0.0)
        alpha, delta = _coefs(stats_ref, b, gu_ref, beu_ref)
        h = jnp.maximum(alpha * x + delta, 0.0)
        e = jnp.dot(wu_ref[...], h, preferred_element_type=jnp.float32)
        e = e + bu_ref[...]
        m = jnp.max(e, axis=0, keepdims=True)
        p = jnp.exp(e - m)
        s2 = p / jnp.sum(p, axis=0, keepdims=True)
        out_ref[0] = (
            jnp.dot(ws1_ref[...], x, preferred_element_type=jnp.float32)
            + jnp.dot(a_ref[...], s2, preferred_element_type=jnp.float32)
            + bs_ref[...])


def kernel(data, bn_d_gamma, bn_d_beta, W_d, b_d, bn_u_gamma, bn_u_beta, W_u,
           b_u, W_g1, b_g1, bn_g1_gamma, bn_g1_beta, W_g2, b_g2, bn_g2_gamma,
           bn_g2_beta, W_s, b_s):
    f32 = jnp.float32
    x3 = data[..., 0]

    s1 = bn_g1_gamma * _BN_S
    s2 = bn_g2_gamma * _BN_S
    wg1t = (W_g1 * s1[:, None]).T
    b1row = (b_g1 * s1 + bn_g1_beta)[None, :]
    wg2t = (W_g2 * s2[:, None]).T
    b2row = (b_g2 * s2 + bn_g2_beta)[None, :]
    m1 = wg1t[:_CL] + wg1t[_CL:]
    a2 = wg1t[_CL:]

    def full(shape):
        nd = len(shape)
        return pl.BlockSpec(shape, lambda p, b, n, _nd=nd: (0,) * _nd)

    tile_spec = pl.BlockSpec((1, _C, _T), lambda p, b, n: (b, 0, n))
    out_spec = pl.BlockSpec(
        (1, _C, _T),
        lambda p, b, n: (jnp.where(p == 2, b, 0), 0, jnp.where(p == 2, n, 0)))

    outp = pl.pallas_call(
        _body,
        grid=(3, _B, _NT),
        in_specs=[
            tile_spec,
            full((_C, 1)), full((_C, 1)), full((_C, 1)), full((_C, 1)),
            full((_CL, _C)), full((_CL, _C)), full((_CL, 1)),
            full((_CL, _CL)), full((_CL, _CL)), full((_CL, _CL)),
            full((1, _CL)), full((1, _CL)),
            full((_C, _C)), full((_C, _C)), full((_C, 1)),
        ],
        out_specs=out_spec,
        out_shape=jax.ShapeDtypeStruct((_B, _C, _N), f32),
        scratch_shapes=[
            pltpu.VMEM((_B, _C, 2), f32),
            pltpu.VMEM((_CL, 1), f32),
            pltpu.VMEM((_CL, 1), f32),
            pltpu.VMEM((_CL, _C), f32),
            pltpu.VMEM((_B, _CL, _C), f32),
            pltpu.VMEM((_C, _CL), f32),
        ],
        compiler_params=pltpu.CompilerParams(
            dimension_semantics=("arbitrary", "arbitrary", "arbitrary")),
    )(x3,
      bn_d_gamma[:, None], bn_d_beta[:, None],
      bn_u_gamma[:, None], bn_u_beta[:, None],
      W_d, W_u, b_u[:, None],
      m1, a2, wg2t, b1row, b2row,
      W_s[:, :_C], W_s[:, _C:], b_s[:, None])

    return outp[..., None]


# speedup vs baseline: 2.3794x; 1.1084x over previous
import jax
import jax.numpy as jnp
from jax.experimental import pallas as pl
from jax.experimental.pallas import tpu as pltpu

_B, _C, _N, _CL, _K = 4, 128, 10000, 256, 6
_T = 10240
_NT = (_N + _T - 1) // _T
_NEG = float("-inf")
_BN_S = 0.9999950000374997


def _lane_mask(nt):
    col = jax.lax.broadcasted_iota(jnp.int32, (_C, _T), 1) + nt * _T
    return col < _N


def _coefs(stats_ref, b, g_ref, be_ref):
    mean = stats_ref[b, :, 0:1] * (1.0 / _N)
    var = stats_ref[b, :, 1:2] * (1.0 / _N) - mean * mean
    a = g_ref[...] * _BN_S * jax.lax.rsqrt(var + 1e-3)
    d = be_ref[...] - a * mean
    return a, d


def _body(x_ref, gd_ref, bd_ref, gu_ref, beu_ref, wd_ref, wu_ref, bu_ref,
          m1_ref, a2_ref, wg2t_ref, b1_ref, b2_ref, ws1_ref, ws2_ref, bs_ref,
          out_ref, stats_ref, m_ref, s_ref, u_ref, xds_ref, a_ref):
    ph = pl.program_id(0)
    b = pl.program_id(1)
    nt = pl.program_id(2)

    @pl.when(ph == 0)
    def _stats():
        x = jnp.where(_lane_mask(nt), x_ref[0], 0.0)
        s0 = jnp.sum(x, axis=1, keepdims=True)
        s1 = jnp.sum(x * x, axis=1, keepdims=True)

        @pl.when(nt == 0)
        def _():
            stats_ref[b, :, 0:1] = s0
            stats_ref[b, :, 1:2] = s1

        @pl.when(nt > 0)
        def _():
            stats_ref[b, :, 0:1] += s0
            stats_ref[b, :, 1:2] += s1

    @pl.when(ph == 1)
    def _pool():
        @pl.when(nt == 0)
        def _():
            m_ref[...] = jnp.full_like(m_ref, _NEG)
            s_ref[...] = jnp.zeros_like(s_ref)
            u_ref[...] = jnp.zeros_like(u_ref)

        valid = _lane_mask(nt)
        x = jnp.where(valid, x_ref[0], 0.0)
        alpha, delta = _coefs(stats_ref, b, gd_ref, bd_ref)
        h = jnp.maximum(alpha * x + delta, 0.0)
        e = jnp.dot(wd_ref[...], h, preferred_element_type=jnp.float32)
        e = jnp.where(jax.lax.broadcasted_iota(jnp.int32, e.shape, 1)
                      + nt * _T < _N, e, _NEG)
        m_old = m_ref[...]
        m_new = jnp.maximum(m_old, jnp.max(e, axis=1, keepdims=True))
        scale = jnp.exp(m_old - m_new)
        p = jnp.exp(e - m_new)
        s_ref[...] = s_ref[...] * scale + jnp.sum(p, axis=1, keepdims=True)
        u_ref[...] = u_ref[...] * scale + jax.lax.dot_general(
            p, x, (((1,), (1,)), ((), ())), preferred_element_type=jnp.float32)
        m_ref[...] = m_new

        @pl.when(nt == _NT - 1)
        def _():
            xds_ref[b] = u_ref[...] / s_ref[...]

    @pl.when(ph == 2)
    def _unpool():
        @pl.when(nt == 0)
        def _dgcnn():
            xv = xds_ref[b]
            g_inner = jax.lax.dot_general(
                xv, xv, (((0,), (0,)), ((), ())),
                preferred_element_type=jnp.float32)
            rows = jax.lax.broadcasted_iota(jnp.int32, (_C, _C), 0)
            cols = jax.lax.broadcasted_iota(jnp.int32, (_C, _C), 1)
            eye = (rows == cols).astype(jnp.float32)
            d_col = jnp.sum(g_inner * eye, axis=1, keepdims=True)
            d_row = jnp.sum(xv * xv, axis=0, keepdims=True)
            pd = 2.0 * g_inner - d_col - d_row
            p_mat = jax.lax.dot_general(
                xv, m1_ref[...], (((0,), (0,)), ((), ())),
                preferred_element_type=jnp.float32)
            q_mat = jax.lax.dot_general(
                xv, a2_ref[...], (((0,), (0,)), ((), ())),
                preferred_element_type=jnp.float32)
            b1 = b1_ref[...]
            b2 = b2_ref[...]
            work = pd
            gmax = jnp.full((_C, _CL), _NEG, jnp.float32)
            for _ in range(_K):
                m = jnp.max(work, axis=1, keepdims=True)
                cand = jnp.where(work == m, cols, jnp.int32(1 << 30))
                sel = jnp.min(cand, axis=1, keepdims=True)
                hit = cols == sel
                onehot = hit.astype(jnp.float32)
                f_q = jnp.dot(onehot, q_mat, preferred_element_type=jnp.float32)
                g1 = jnp.maximum(p_mat - f_q + b1, 0.0)
                g2 = jnp.maximum(
                    jnp.dot(g1, wg2t_ref[...],
                            preferred_element_type=jnp.float32) + b2, 0.0)
                gmax = jnp.maximum(gmax, g2)
                work = jnp.where(hit, _NEG, work)
            a_ref[...] = jnp.dot(ws2_ref[...], gmax,
                                 preferred_element_type=jnp.float32)

        x = jnp.where(_lane_mask(nt), x_ref[0], 0.0)
        alpha, delta = _coefs(stats_ref, b, gu_ref, beu_ref)
        h = jnp.maximum(alpha * x + delta, 0.0)
        e = jnp.dot(wu_ref[...], h, preferred_element_type=jnp.float32)
        e = e + bu_ref[...]
        m = jnp.max(e, axis=0, keepdims=True)
        p = jnp.exp(e - m)
        s2 = p / jnp.sum(p, axis=0, keepdims=True)
        out_ref[0] = (
            jnp.dot(ws1_ref[...], x, preferred_element_type=jnp.float32)
            + jnp.dot(a_ref[...], s2, preferred_element_type=jnp.float32)
            + bs_ref[...])


def kernel(data, bn_d_gamma, bn_d_beta, W_d, b_d, bn_u_gamma, bn_u_beta, W_u,
           b_u, W_g1, b_g1, bn_g1_gamma, bn_g1_beta, W_g2, b_g2, bn_g2_gamma,
           bn_g2_beta, W_s, b_s):
    f32 = jnp.float32
    x3 = data[..., 0]

    s1 = bn_g1_gamma * _BN_S
    s2 = bn_g2_gamma * _BN_S
    wg1t = (W_g1 * s1[:, None]).T
    b1row = (b_g1 * s1 + bn_g1_beta)[None, :]
    wg2t = (W_g2 * s2[:, None]).T
    b2row = (b_g2 * s2 + bn_g2_beta)[None, :]
    m1 = wg1t[:_CL] + wg1t[_CL:]
    a2 = wg1t[_CL:]

    def full(shape):
        nd = len(shape)
        return pl.BlockSpec(shape, lambda p, b, n, _nd=nd: (0,) * _nd)

    tile_spec = pl.BlockSpec((1, _C, _T), lambda p, b, n: (b, 0, n))
    out_spec = pl.BlockSpec(
        (1, _C, _T),
        lambda p, b, n: (jnp.where(p == 2, b, 0), 0, jnp.where(p == 2, n, 0)))

    outp = pl.pallas_call(
        _body,
        grid=(3, _B, _NT),
        in_specs=[
            tile_spec,
            full((_C, 1)), full((_C, 1)), full((_C, 1)), full((_C, 1)),
            full((_CL, _C)), full((_CL, _C)), full((_CL, 1)),
            full((_CL, _CL)), full((_CL, _CL)), full((_CL, _CL)),
            full((1, _CL)), full((1, _CL)),
            full((_C, _C)), full((_C, _C)), full((_C, 1)),
        ],
        out_specs=out_spec,
        out_shape=jax.ShapeDtypeStruct((_B, _C, _N), f32),
        scratch_shapes=[
            pltpu.VMEM((_B, _C, 2), f32),
            pltpu.VMEM((_CL, 1), f32),
            pltpu.VMEM((_CL, 1), f32),
            pltpu.VMEM((_CL, _C), f32),
            pltpu.VMEM((_B, _CL, _C), f32),
            pltpu.VMEM((_C, _CL), f32),
        ],
        compiler_params=pltpu.CompilerParams(
            dimension_semantics=("arbitrary", "arbitrary", "arbitrary")),
    )(x3,
      bn_d_gamma[:, None], bn_d_beta[:, None],
      bn_u_gamma[:, None], bn_u_beta[:, None],
      W_d, W_u, b_u[:, None],
      m1, a2, wg2t, b1row, b2row,
      W_s[:, :_C], W_s[:, _C:], b_s[:, None])

    return outp[..., None]


# speedup vs baseline: 2.4135x; 1.0144x over previous
import jax
import jax.numpy as jnp
from jax.experimental import pallas as pl
from jax.experimental.pallas import tpu as pltpu

_B, _C, _N, _CL, _K = 4, 128, 10000, 256, 6
_NEG = float("-inf")
_BN_S = 0.9999950000374997


def _affine(x, s0, s1, g_ref, be_ref):
    mean = s0 * (1.0 / _N)
    var = s1 * (1.0 / _N) - mean * mean
    a = g_ref[...] * _BN_S * jax.lax.rsqrt(var + 1e-3)
    d = be_ref[...] - a * mean
    return jnp.maximum(a * x + d, 0.0)


def _body(x_ref, gd_ref, bd_ref, gu_ref, beu_ref, wd_ref, wu_ref, bu_ref,
          m1_ref, a2_ref, wg2t_ref, b1_ref, b2_ref, ws1_ref, ws2_ref, bs_ref,
          out_ref):
    x = x_ref[0]
    s0 = jnp.sum(x, axis=1, keepdims=True)
    s1 = jnp.sum(x * x, axis=1, keepdims=True)

    h = _affine(x, s0, s1, gd_ref, bd_ref)
    e = jnp.dot(wd_ref[...], h, preferred_element_type=jnp.float32)
    p = jnp.exp(e - jnp.max(e, axis=1, keepdims=True))
    xv = jax.lax.dot_general(
        p, x, (((1,), (1,)), ((), ())), preferred_element_type=jnp.float32)
    xv = xv / jnp.sum(p, axis=1, keepdims=True)

    g_inner = jax.lax.dot_general(
        xv, xv, (((0,), (0,)), ((), ())), preferred_element_type=jnp.float32)
    rows = jax.lax.broadcasted_iota(jnp.int32, (_C, _C), 0)
    cols = jax.lax.broadcasted_iota(jnp.int32, (_C, _C), 1)
    eye = (rows == cols).astype(jnp.float32)
    d_col = jnp.sum(g_inner * eye, axis=1, keepdims=True)
    d_row = jnp.sum(xv * xv, axis=0, keepdims=True)
    pd = 2.0 * g_inner - d_col - d_row
    p_mat = jax.lax.dot_general(
        xv, m1_ref[...], (((0,), (0,)), ((), ())),
        preferred_element_type=jnp.float32)
    q_mat = jax.lax.dot_general(
        xv, a2_ref[...], (((0,), (0,)), ((), ())),
        preferred_element_type=jnp.float32)
    b1 = b1_ref[...]
    b2 = b2_ref[...]
    work = pd
    gmax = jnp.full((_C, _CL), _NEG, jnp.float32)
    for _ in range(_K):
        m = jnp.max(work, axis=1, keepdims=True)
        cand = jnp.where(work == m, cols, jnp.int32(1 << 30))
        sel = jnp.min(cand, axis=1, keepdims=True)
        hit = cols == sel
        onehot = hit.astype(jnp.float32)
        f_q = jnp.dot(onehot, q_mat, preferred_element_type=jnp.float32)
        g1 = jnp.maximum(p_mat - f_q + b1, 0.0)
        g2 = jnp.maximum(
            jnp.dot(g1, wg2t_ref[...], preferred_element_type=jnp.float32)
            + b2, 0.0)
        gmax = jnp.maximum(gmax, g2)
        work = jnp.where(hit, _NEG, work)
    a_mat = jnp.dot(ws2_ref[...], gmax,
                    preferred_element_type=jnp.float32)

    h2 = _affine(x, s0, s1, gu_ref, beu_ref)
    e2 = jnp.dot(wu_ref[...], h2, preferred_element_type=jnp.float32)
    e2 = e2 + bu_ref[...]
    p2 = jnp.exp(e2 - jnp.max(e2, axis=0, keepdims=True))
    s2n = p2 / jnp.sum(p2, axis=0, keepdims=True)
    out_ref[0] = (
        jnp.dot(ws1_ref[...], x, preferred_element_type=jnp.float32)
        + jnp.dot(a_mat, s2n, preferred_element_type=jnp.float32)
        + bs_ref[...])


def kernel(data, bn_d_gamma, bn_d_beta, W_d, b_d, bn_u_gamma, bn_u_beta, W_u,
           b_u, W_g1, b_g1, bn_g1_gamma, bn_g1_beta, W_g2, b_g2, bn_g2_gamma,
           bn_g2_beta, W_s, b_s):
    f32 = jnp.float32
    x3 = data[..., 0]

    s1 = bn_g1_gamma * _BN_S
    s2 = bn_g2_gamma * _BN_S
    wg1t = (W_g1 * s1[:, None]).T
    b1row = (b_g1 * s1 + bn_g1_beta)[None, :]
    wg2t = (W_g2 * s2[:, None]).T
    b2row = (b_g2 * s2 + bn_g2_beta)[None, :]
    m1 = wg1t[:_CL] + wg1t[_CL:]
    a2 = wg1t[_CL:]

    def full(shape):
        nd = len(shape)
        return pl.BlockSpec(shape, lambda b, _nd=nd: (0,) * _nd)

    tile_spec = pl.BlockSpec((1, _C, _N), lambda b: (b, 0, 0))

    outp = pl.pallas_call(
        _body,
        grid=(_B,),
        in_specs=[
            tile_spec,
            full((_C, 1)), full((_C, 1)), full((_C, 1)), full((_C, 1)),
            full((_CL, _C)), full((_CL, _C)), full((_CL, 1)),
            full((_CL, _CL)), full((_CL, _CL)), full((_CL, _CL)),
            full((1, _CL)), full((1, _CL)),
            full((_C, _C)), full((_C, _C)), full((_C, 1)),
        ],
        out_specs=tile_spec,
        out_shape=jax.ShapeDtypeStruct((_B, _C, _N), f32),
        compiler_params=pltpu.CompilerParams(
            dimension_semantics=("arbitrary",)),
    )(x3,
      bn_d_gamma[:, None], bn_d_beta[:, None],
      bn_u_gamma[:, None], bn_u_beta[:, None],
      W_d, W_u, b_u[:, None],
      m1, a2, wg2t, b1row, b2row,
      W_s[:, :_C], W_s[:, _C:], b_s[:, None])

    return outp[..., None]


# speedup vs baseline: 2.4988x; 1.0353x over previous
import jax
import jax.numpy as jnp
from jax.experimental import pallas as pl
from jax.experimental.pallas import tpu as pltpu

_B, _C, _N, _CL, _K = 4, 128, 10000, 256, 6
_NEG = float("-inf")
_BN_S = 0.9999950000374997


def _affine(x, s0, s1, g_ref, be_ref):
    mean = s0 * (1.0 / _N)
    var = s1 * (1.0 / _N) - mean * mean
    a = g_ref[...] * _BN_S * jax.lax.rsqrt(var + 1e-3)
    d = be_ref[...] - a * mean
    return jnp.maximum(a * x + d, 0.0)


_TC = 2048
_CHUNKS = [(o, min(_TC, _N - o)) for o in range(0, _N, _TC)]


def _body(x_ref, gd_ref, bd_ref, gu_ref, beu_ref, wd_ref, wu_ref, bu_ref,
          m1_ref, a2_ref, wg2t_ref, b1_ref, b2_ref, ws1_ref, ws2_ref, bs_ref,
          out_ref):
    s0 = jnp.zeros((_C, 1), jnp.float32)
    s1 = jnp.zeros((_C, 1), jnp.float32)
    for o, w in _CHUNKS:
        xc = x_ref[0, :, pl.ds(o, w)]
        s0 = s0 + jnp.sum(xc, axis=1, keepdims=True)
        s1 = s1 + jnp.sum(xc * xc, axis=1, keepdims=True)

    mx = jnp.full((_CL, 1), _NEG, jnp.float32)
    se = jnp.zeros((_CL, 1), jnp.float32)
    u = jnp.zeros((_CL, _C), jnp.float32)
    for o, w in _CHUNKS:
        xc = x_ref[0, :, pl.ds(o, w)]
        h = _affine(xc, s0, s1, gd_ref, bd_ref)
        e = jnp.dot(wd_ref[...], h, preferred_element_type=jnp.float32)
        m_new = jnp.maximum(mx, jnp.max(e, axis=1, keepdims=True))
        sc = jnp.exp(mx - m_new)
        p = jnp.exp(e - m_new)
        se = se * sc + jnp.sum(p, axis=1, keepdims=True)
        u = u * sc + jax.lax.dot_general(
            p, xc, (((1,), (1,)), ((), ())),
            preferred_element_type=jnp.float32)
        mx = m_new
    xv = u / se

    g_inner = jax.lax.dot_general(
        xv, xv, (((0,), (0,)), ((), ())), preferred_element_type=jnp.float32)
    rows = jax.lax.broadcasted_iota(jnp.int32, (_C, _C), 0)
    cols = jax.lax.broadcasted_iota(jnp.int32, (_C, _C), 1)
    eye = (rows == cols).astype(jnp.float32)
    d_col = jnp.sum(g_inner * eye, axis=1, keepdims=True)
    d_row = jnp.sum(xv * xv, axis=0, keepdims=True)
    pd = 2.0 * g_inner - d_col - d_row
    p_mat = jax.lax.dot_general(
        xv, m1_ref[...], (((0,), (0,)), ((), ())),
        preferred_element_type=jnp.float32)
    q_mat = jax.lax.dot_general(
        xv, a2_ref[...], (((0,), (0,)), ((), ())),
        preferred_element_type=jnp.float32)
    b1 = b1_ref[...]
    b2 = b2_ref[...]
    work = pd
    gmax = jnp.full((_C, _CL), _NEG, jnp.float32)
    for _ in range(_K):
        m = jnp.max(work, axis=1, keepdims=True)
        cand = jnp.where(work == m, cols, jnp.int32(1 << 30))
        sel = jnp.min(cand, axis=1, keepdims=True)
        hit = cols == sel
        onehot = hit.astype(jnp.float32)
        f_q = jnp.dot(onehot, q_mat, preferred_element_type=jnp.float32)
        g1 = jnp.maximum(p_mat - f_q + b1, 0.0)
        g2 = jnp.maximum(
            jnp.dot(g1, wg2t_ref[...], preferred_element_type=jnp.float32)
            + b2, 0.0)
        gmax = jnp.maximum(gmax, g2)
        work = jnp.where(hit, _NEG, work)
    a_mat = jnp.dot(ws2_ref[...], gmax,
                    preferred_element_type=jnp.float32)

    for o, w in _CHUNKS:
        xc = x_ref[0, :, pl.ds(o, w)]
        h2 = _affine(xc, s0, s1, gu_ref, beu_ref)
        e2 = jnp.dot(wu_ref[...], h2, preferred_element_type=jnp.float32)
        e2 = e2 + bu_ref[...]
        p2 = jnp.exp(e2 - jnp.max(e2, axis=0, keepdims=True))
        s2n = p2 / jnp.sum(p2, axis=0, keepdims=True)
        out_ref[0, :, pl.ds(o, w)] = (
            jnp.dot(ws1_ref[...], xc, preferred_element_type=jnp.float32)
            + jnp.dot(a_mat, s2n, preferred_element_type=jnp.float32)
            + bs_ref[...])


def kernel(data, bn_d_gamma, bn_d_beta, W_d, b_d, bn_u_gamma, bn_u_beta, W_u,
           b_u, W_g1, b_g1, bn_g1_gamma, bn_g1_beta, W_g2, b_g2, bn_g2_gamma,
           bn_g2_beta, W_s, b_s):
    f32 = jnp.float32
    x3 = data[..., 0]

    s1 = bn_g1_gamma * _BN_S
    s2 = bn_g2_gamma * _BN_S
    wg1t = (W_g1 * s1[:, None]).T
    b1row = (b_g1 * s1 + bn_g1_beta)[None, :]
    wg2t = (W_g2 * s2[:, None]).T
    b2row = (b_g2 * s2 + bn_g2_beta)[None, :]
    m1 = wg1t[:_CL] + wg1t[_CL:]
    a2 = wg1t[_CL:]

    def full(shape):
        nd = len(shape)
        return pl.BlockSpec(shape, lambda b, _nd=nd: (0,) * _nd)

    tile_spec = pl.BlockSpec((1, _C, _N), lambda b: (b, 0, 0))

    outp = pl.pallas_call(
        _body,
        grid=(_B,),
        in_specs=[
            tile_spec,
            full((_C, 1)), full((_C, 1)), full((_C, 1)), full((_C, 1)),
            full((_CL, _C)), full((_CL, _C)), full((_CL, 1)),
            full((_CL, _CL)), full((_CL, _CL)), full((_CL, _CL)),
            full((1, _CL)), full((1, _CL)),
            full((_C, _C)), full((_C, _C)), full((_C, 1)),
        ],
        out_specs=tile_spec,
        out_shape=jax.ShapeDtypeStruct((_B, _C, _N), f32),
        compiler_params=pltpu.CompilerParams(
            dimension_semantics=("arbitrary",)),
    )(x3,
      bn_d_gamma[:, None], bn_d_beta[:, None],
      bn_u_gamma[:, None], bn_u_beta[:, None],
      W_d, W_u, b_u[:, None],
      m1, a2, wg2t, b1row, b2row,
      W_s[:, :_C], W_s[:, _C:], b_s[:, None])

    return outp[..., None]
